# baseline (speedup 1.0000x reference)
import functools
import math

import jax
import jax.numpy as jnp
from jax import lax
from jax.experimental import pallas as pl
from jax.experimental.pallas import tpu as pltpu
from jax.experimental.pallas import tpu_sc as plsc

N = 10000
E = 320000
H = 128
AVG_LOG = math.log(33.0)

BE = 1600
BN = 1000
BS = 2000


_SC_NC = 2
_SC_NS = 16
_NW = _SC_NC * _SC_NS


@functools.lru_cache(maxsize=None)
def _sc_gather(Mrows, Q):
    qpw = Q // _NW
    CH = 400
    assert qpw % CH == 0 and Q % _NW == 0
    mesh = plsc.VectorSubcoreMesh(core_axis_name="c", subcore_axis_name="s")

    @functools.partial(
        pl.kernel,
        mesh=mesh,
        out_type=jax.ShapeDtypeStruct((Q, H), jnp.float32),
        scratch_types=[
            pltpu.VMEM((CH,), jnp.int32),
            pltpu.VMEM((CH, H), jnp.float32),
            pltpu.SemaphoreType.DMA,
        ],
    )
    def gk(table_hbm, idx_hbm, out_hbm, idx_v, rows_v, sem):
        wid = lax.axis_index("s") * _SC_NC + lax.axis_index("c")
        w0 = pl.multiple_of(wid * qpw, 8)

        def body(c, carry):
            base = pl.multiple_of(w0 + c * CH, 8)
            pltpu.sync_copy(idx_hbm.at[pl.ds(base, CH)], idx_v)
            pltpu.async_copy(table_hbm.at[idx_v], rows_v, sem).wait()
            pltpu.sync_copy(rows_v, out_hbm.at[pl.ds(base, CH)])
            return carry

        lax.fori_loop(0, qpw // CH, body, 0)

    return gk


def _gather(table, idx):
    return _sc_gather(table.shape[0], idx.shape[0])(table, idx)


def _dot(a, b):
    return jnp.dot(a, b, preferred_element_type=jnp.float32)


def _prep_kernel(eeW_ref, eeb_ref, preW_ref, preb_ref, W3p_ref, bf_ref):
    for i in range(2):
        W3 = preW_ref[i][2 * H:, :]
        W3p_ref[i] = _dot(eeW_ref[i], W3)
        bf_ref[i] = preb_ref[i] + _dot(eeb_ref[i], W3)


def _prep(ee_W, ee_b, pre_W, pre_b):
    return pl.pallas_call(
        _prep_kernel,
        out_shape=(
            jax.ShapeDtypeStruct((2, H, H), jnp.float32),
            jax.ShapeDtypeStruct((2, 1, H), jnp.float32),
        ),
    )(ee_W, ee_b.reshape(2, 1, H), pre_W, pre_b.reshape(2, 1, H))


def _encN_kernel(x_ref, W_ref, b_ref, o_ref):
    o_ref[...] = _dot(x_ref[...], W_ref[...]) + b_ref[...]


def _enc_nodes(x, W, b):
    return pl.pallas_call(
        _encN_kernel,
        grid=(N // BN,),
        in_specs=[
            pl.BlockSpec((BN, H), lambda i: (i, 0)),
            pl.BlockSpec((H, H), lambda i: (0, 0)),
            pl.BlockSpec((1, H), lambda i: (0, 0)),
        ],
        out_specs=pl.BlockSpec((BN, H), lambda i: (i, 0)),
        out_shape=jax.ShapeDtypeStruct((N, H), jnp.float32),
    )(x, W, b.reshape(1, H))


def _enc_edges(ea, W, b):
    D = ea.shape[1]
    return pl.pallas_call(
        _encN_kernel,
        grid=(E // BE,),
        in_specs=[
            pl.BlockSpec((BE, D), lambda i: (i, 0)),
            pl.BlockSpec((D, H), lambda i: (0, 0)),
            pl.BlockSpec((1, H), lambda i: (0, 0)),
        ],
        out_specs=pl.BlockSpec((BE, H), lambda i: (i, 0)),
        out_shape=jax.ShapeDtypeStruct((E, H), jnp.float32),
    )(ea, W, b.reshape(1, H))


def _tables_kernel(x_ref, W_ref, o_ref):
    o_ref[0] = _dot(x_ref[...], W_ref[0])


def _tables(x, Wstack):
    S = Wstack.shape[0]
    out = pl.pallas_call(
        _tables_kernel,
        grid=(S, N // BN),
        in_specs=[
            pl.BlockSpec((BN, H), lambda s, k: (k, 0)),
            pl.BlockSpec((1, H, H), lambda s, k: (s, 0, 0)),
        ],
        out_specs=pl.BlockSpec((1, BN, H), lambda s, k: (s, k, 0)),
        out_shape=jax.ShapeDtypeStruct((S, N, H), jnp.float32),
    )(x, Wstack)
    return out.reshape(S * N, H)


def _mker_kernel(gd_ref, gs_ref, ea_ref, W3p_ref, bf_ref, m_ref):
    m_ref[...] = (gd_ref[...] + gs_ref[...]
                  + _dot(ea_ref[...], W3p_ref[...]) + bf_ref[...])


def _m_edges(g, ea, W3p, bf):
    KB = E // BE
    return pl.pallas_call(
        _mker_kernel,
        grid=(KB,),
        in_specs=[
            pl.BlockSpec((BE, H), lambda i: (i, 0)),
            pl.BlockSpec((BE, H), lambda i: (i + KB, 0)),
            pl.BlockSpec((BE, H), lambda i: (i, 0)),
            pl.BlockSpec((H, H), lambda i: (0, 0)),
            pl.BlockSpec((1, H), lambda i: (0, 0)),
        ],
        out_specs=pl.BlockSpec((BE, H), lambda i: (i, 0)),
        out_shape=jax.ShapeDtypeStruct((E, H), jnp.float32),
    )(g, g, ea, W3p, bf)


def _scatter_kernel(m_ref, idx_ref, acc_ref, mm_ref):
    @pl.when(pl.program_id(0) == 0)
    def _init():
        acc_ref[...] = jnp.zeros((N, 3 * H), jnp.float32)
        mm_ref[...] = jnp.full((N, 2 * H), -jnp.inf, jnp.float32)

    ones = jnp.ones((1, H), jnp.float32)

    def body(j, carry):
        d = idx_ref[0, 0, j]
        row = m_ref[pl.ds(j, 1), :]
        r3 = jnp.concatenate([row, row * row, ones], axis=1)
        r2 = jnp.concatenate([row, -row], axis=1)
        acc_ref[pl.ds(d, 1), :] = acc_ref[pl.ds(d, 1), :] + r3
        mm_ref[pl.ds(d, 1), :] = jnp.maximum(mm_ref[pl.ds(d, 1), :], r2)
        return carry

    lax.fori_loop(0, BS, body, 0)


def _scatter(m, dst3):
    return pl.pallas_call(
        _scatter_kernel,
        grid=(E // BS,),
        in_specs=[
            pl.BlockSpec((BS, H), lambda i: (i, 0)),
            pl.BlockSpec((1, 1, BS), lambda i: (i, 0, 0), memory_space=pltpu.SMEM),
        ],
        out_specs=[
            pl.BlockSpec((N, 3 * H), lambda i: (0, 0)),
            pl.BlockSpec((N, 2 * H), lambda i: (0, 0)),
        ],
        out_shape=[
            jax.ShapeDtypeStruct((N, 3 * H), jnp.float32),
            jax.ShapeDtypeStruct((N, 2 * H), jnp.float32),
        ],
    )(m, dst3)


def _nodeA_kernel(acc_ref, mm_ref, x_ref, PW_ref,
                  pb_ref, lin_ref, lb_ref, out_ref, cs_ref, cq_ref):
    acc = acc_ref[...]
    mm = mm_ref[...]
    s = acc[:, :H]
    s2 = acc[:, H:2 * H]
    cnt = acc[:, 2 * H:]
    deg = jnp.maximum(cnt, 1.0)
    has = cnt > 0.0
    mean = s / deg
    std = jnp.sqrt(jax.nn.relu(s2 / deg - mean * mean) + 1e-5)
    mn = jnp.where(has, -mm[:, H:], 0.0)
    mx = jnp.where(has, mm[:, :H], 0.0)
    logd = jnp.log(deg + 1.0)
    amp = logd * (1.0 / AVG_LOG)
    att = AVG_LOG / logd
    A = (mean, mn, mx, std)
    t0 = _dot(x_ref[...], PW_ref[0])
    t1 = sum(_dot(A[k], PW_ref[1 + k]) for k in range(4))
    t2 = sum(_dot(A[k], PW_ref[5 + k]) for k in range(4))
    t3 = sum(_dot(A[k], PW_ref[9 + k]) for k in range(4))
    out = t0 + t1 + amp * t2 + att * t3 + pb_ref[...]
    out = _dot(out, lin_ref[...]) + lb_ref[...]
    out_ref[...] = out

    @pl.when(pl.program_id(0) == 0)
    def _init():
        cs_ref[...] = jnp.zeros((1, H), jnp.float32)
        cq_ref[...] = jnp.zeros((1, H), jnp.float32)

    cs_ref[...] = cs_ref[...] + jnp.sum(out, axis=0, keepdims=True)
    cq_ref[...] = cq_ref[...] + jnp.sum(out * out, axis=0, keepdims=True)


def _nodeA(acc, mm, x, PW, pb, lin, lb):
    blk = pl.BlockSpec((BN, H), lambda i: (i, 0))
    one = pl.BlockSpec((1, H), lambda i: (0, 0))
    return pl.pallas_call(
        _nodeA_kernel,
        grid=(N // BN,),
        in_specs=[pl.BlockSpec((BN, 3 * H), lambda i: (i, 0)),
                  pl.BlockSpec((BN, 2 * H), lambda i: (i, 0)),
                  blk,
                  pl.BlockSpec((13, H, H), lambda i: (0, 0, 0)),
                  one,
                  pl.BlockSpec((H, H), lambda i: (0, 0)),
                  one],
        out_specs=[blk, one, one],
        out_shape=[
            jax.ShapeDtypeStruct((N, H), jnp.float32),
            jax.ShapeDtypeStruct((1, H), jnp.float32),
            jax.ShapeDtypeStruct((1, H), jnp.float32),
        ],
    )(acc, mm, x, PW, pb.reshape(1, H), lin, lb.reshape(1, H))


def _nodeB_kernel(out_ref, cs_ref, cq_ref, x_ref, bw_ref, bb_ref, xn_ref):
    mu = cs_ref[...] * (1.0 / N)
    var = cq_ref[...] * (1.0 / N) - mu * mu
    inv = jax.lax.rsqrt(var + 1e-5)
    bn = (out_ref[...] - mu) * inv * bw_ref[...] + bb_ref[...]
    xn_ref[...] = (x_ref[...] + jax.nn.relu(bn)) * 0.5


def _nodeB(out, cs, cq, x, bw, bb):
    blk = pl.BlockSpec((BN, H), lambda i: (i, 0))
    one = pl.BlockSpec((1, H), lambda i: (0, 0))
    return pl.pallas_call(
        _nodeB_kernel,
        grid=(N // BN,),
        in_specs=[blk, one, one, blk, one, one],
        out_specs=blk,
        out_shape=jax.ShapeDtypeStruct((N, H), jnp.float32),
    )(out, cs, cq, x, bw.reshape(1, H), bb.reshape(1, H))


def _em_kernel(ga_ref, gb_ref, ea_ref, B3_ref, b1_ref, W2_ref, b2_ref, o_ref):
    hid = (ga_ref[...] + gb_ref[...]
           + _dot(ea_ref[...], B3_ref[...]) + b1_ref[...])
    em = _dot(jax.nn.relu(hid), W2_ref[...]) + b2_ref[...]
    o_ref[...] = ea_ref[...] + em * 0.5


def _em_edges(g, off_a, off_b, ea, B3, b1, W2, b2):
    KB = E // BE
    return pl.pallas_call(
        _em_kernel,
        grid=(KB,),
        in_specs=[
            pl.BlockSpec((BE, H), lambda i, o=off_a: (i + o * KB, 0)),
            pl.BlockSpec((BE, H), lambda i, o=off_b: (i + o * KB, 0)),
            pl.BlockSpec((BE, H), lambda i: (i, 0)),
            pl.BlockSpec((H, H), lambda i: (0, 0)),
            pl.BlockSpec((1, H), lambda i: (0, 0)),
            pl.BlockSpec((H, H), lambda i: (0, 0)),
            pl.BlockSpec((1, H), lambda i: (0, 0)),
        ],
        out_specs=pl.BlockSpec((BE, H), lambda i: (i, 0)),
        out_shape=jax.ShapeDtypeStruct((E, H), jnp.float32),
    )(g, g, ea, B3, b1.reshape(1, H), W2, b2.reshape(1, H))


def kernel(x, edge_index, edge_attr, node_W, node_b, edge_W, edge_b, ee_W,
           ee_b, pre_W, pre_b, post_W, post_b, lin_W, lin_b, em1_W, em1_b,
           em2_W, em2_b, bn_w, bn_b):
    src = edge_index[0]
    dst = edge_index[1]
    dst3 = dst.reshape(E // BS, 1, BS)

    W3p, bf = _prep(ee_W, ee_b, pre_W, pre_b)

    xc = _enc_nodes(x, node_W, node_b)
    ea = _enc_edges(edge_attr, edge_W, edge_b)

    W1 = [pre_W[i][:H] for i in range(2)]
    W2 = [pre_W[i][H:2 * H] for i in range(2)]
    B1 = [em1_W[i][:H] for i in range(2)]
    B2 = [em1_W[i][H:2 * H] for i in range(2)]
    B3 = [em1_W[i][2 * H:] for i in range(2)]
    PW = [post_W[i].reshape(13, H, H) for i in range(2)]

    idx_ds = jnp.concatenate([dst, src + N])
    idx_emn = jnp.concatenate([src, dst + N, dst + 2 * N, src + 3 * N])
    idx_em = jnp.concatenate([src, dst + N])

    for i in range(2):
        if i == 0:
            T = _tables(xc, jnp.stack([W1[0], W2[0]]))
            g = _gather(T, idx_ds)
            m = _m_edges(g, ea, W3p[i], bf[i])
        acc, mm = _scatter(m, dst3)
        out, cs, cq = _nodeA(acc, mm, xc, PW[i], post_b[i],
                             lin_W[i], lin_b[i])
        xc = _nodeB(out, cs, cq, xc, bn_w[i], bn_b[i])
        if i == 0:
            T = _tables(xc, jnp.stack([B1[0], B2[0], W1[1], W2[1]]))
            g = _gather(T, idx_emn)
            ea = _em_edges(g, 0, 1, ea, B3[0], em1_b[0], em2_W[0], em2_b[0])
            m = _m_edges(g[2 * E:], ea, W3p[1], bf[1])
        else:
            T = _tables(xc, jnp.stack([B1[1], B2[1]]))
            g = _gather(T, idx_em)
            ea = _em_edges(g, 0, 1, ea, B3[1], em1_b[1], em2_W[1], em2_b[1])
    return xc, ea


# speedup vs baseline: 1.4810x; 1.4810x over previous
import functools
import math

import jax
import jax.numpy as jnp
from jax import lax
from jax.experimental import pallas as pl
from jax.experimental.pallas import tpu as pltpu
from jax.experimental.pallas import tpu_sc as plsc

N = 10000
E = 320000
H = 128
AVG_LOG = math.log(33.0)

BE = 1600
BN = 1000
BS = 2000


_SC_NC = 2
_SC_NS = 16
_NW = _SC_NC * _SC_NS


@functools.lru_cache(maxsize=None)
def _sc_gather(Mrows, Q):
    qpw = Q // _NW
    CH = 400
    assert qpw % CH == 0 and Q % _NW == 0
    mesh = plsc.VectorSubcoreMesh(core_axis_name="c", subcore_axis_name="s")

    @functools.partial(
        pl.kernel,
        mesh=mesh,
        out_type=jax.ShapeDtypeStruct((Q, H), jnp.float32),
        scratch_types=[
            pltpu.VMEM((CH,), jnp.int32),
            pltpu.VMEM((CH, H), jnp.float32),
            pltpu.SemaphoreType.DMA,
        ],
    )
    def gk(table_hbm, idx_hbm, out_hbm, idx_v, rows_v, sem):
        wid = lax.axis_index("s") * _SC_NC + lax.axis_index("c")
        w0 = pl.multiple_of(wid * qpw, 8)

        def body(c, carry):
            base = pl.multiple_of(w0 + c * CH, 8)
            pltpu.sync_copy(idx_hbm.at[pl.ds(base, CH)], idx_v)
            pltpu.async_copy(table_hbm.at[idx_v], rows_v, sem).wait()
            pltpu.sync_copy(rows_v, out_hbm.at[pl.ds(base, CH)])
            return carry

        lax.fori_loop(0, qpw // CH, body, 0)

    return gk


def _gather(table, idx):
    return _sc_gather(table.shape[0], idx.shape[0])(table, idx)


_SCH = 80
_NP = 10240
_ROWS_T = _NP // _SC_NS


@functools.lru_cache(maxsize=None)
def _sc_scatter_add(D):
    epc = E // _NW
    mesh = plsc.VectorSubcoreMesh(core_axis_name="c", subcore_axis_name="s")
    zrows = 128

    @functools.partial(
        pl.kernel,
        mesh=mesh,
        out_type=jax.ShapeDtypeStruct((2 * _NP, D), jnp.float32),
        scratch_types=[
            pltpu.VMEM_SHARED((_NP, D), jnp.float32),
            pltpu.VMEM((_SCH,), jnp.int32),
            pltpu.VMEM((_SCH, D), jnp.float32),
            pltpu.VMEM((zrows, D), jnp.float32),
        ],
    )
    def sk(*refs):
        vals_hbm, idx_hbm, out_hbm, acc, idx_v, vals_v, zero_v = refs
        c = lax.axis_index("c")
        s = lax.axis_index("s")

        zbase = s * _ROWS_T
        for zc in range(_ROWS_T // zrows):
            zero_v[...] = jnp.zeros((zrows, D), jnp.float32)
            pltpu.sync_copy(zero_v, acc.at[pl.ds(zbase + zc * zrows, zrows)])
        plsc.subcore_barrier()

        ebase = c * (E // 2) + s * epc

        def body(k, carry):
            base = pl.multiple_of(ebase + k * _SCH, 8)
            pltpu.sync_copy(idx_hbm.at[pl.ds(base, _SCH)], idx_v)
            pltpu.sync_copy(vals_hbm.at[pl.ds(base, _SCH)], vals_v)
            pltpu.sync_copy(vals_v, acc.at[idx_v], add=True)
            return carry

        lax.fori_loop(0, epc // _SCH, body, 0)
        plsc.subcore_barrier()
        obase = c * _NP + s * _ROWS_T
        pltpu.sync_copy(acc.at[pl.ds(s * _ROWS_T, _ROWS_T)],
                        out_hbm.at[pl.ds(obase, _ROWS_T)])

    return sk


def _scatter_add(vals, idx):
    out = _sc_scatter_add(vals.shape[1])(vals, idx)
    return jnp.concatenate([out[:N], out[_NP:_NP + N]], axis=0)


def _dot(a, b):
    return jnp.dot(a, b, preferred_element_type=jnp.float32)


def _prep_kernel(eeW_ref, eeb_ref, preW_ref, preb_ref, W3p_ref, bf_ref):
    for i in range(2):
        W3 = preW_ref[i][2 * H:, :]
        W3p_ref[i] = _dot(eeW_ref[i], W3)
        bf_ref[i] = preb_ref[i] + _dot(eeb_ref[i], W3)


def _prep(ee_W, ee_b, pre_W, pre_b):
    return pl.pallas_call(
        _prep_kernel,
        out_shape=(
            jax.ShapeDtypeStruct((2, H, H), jnp.float32),
            jax.ShapeDtypeStruct((2, 1, H), jnp.float32),
        ),
    )(ee_W, ee_b.reshape(2, 1, H), pre_W, pre_b.reshape(2, 1, H))


def _encN_kernel(x_ref, W_ref, b_ref, o_ref):
    o_ref[...] = _dot(x_ref[...], W_ref[...]) + b_ref[...]


def _enc_nodes(x, W, b):
    return pl.pallas_call(
        _encN_kernel,
        grid=(N // BN,),
        in_specs=[
            pl.BlockSpec((BN, H), lambda i: (i, 0)),
            pl.BlockSpec((H, H), lambda i: (0, 0)),
            pl.BlockSpec((1, H), lambda i: (0, 0)),
        ],
        out_specs=pl.BlockSpec((BN, H), lambda i: (i, 0)),
        out_shape=jax.ShapeDtypeStruct((N, H), jnp.float32),
    )(x, W, b.reshape(1, H))


def _enc_edges(ea, W, b):
    D = ea.shape[1]
    return pl.pallas_call(
        _encN_kernel,
        grid=(E // BE,),
        in_specs=[
            pl.BlockSpec((BE, D), lambda i: (i, 0)),
            pl.BlockSpec((D, H), lambda i: (0, 0)),
            pl.BlockSpec((1, H), lambda i: (0, 0)),
        ],
        out_specs=pl.BlockSpec((BE, H), lambda i: (i, 0)),
        out_shape=jax.ShapeDtypeStruct((E, H), jnp.float32),
    )(ea, W, b.reshape(1, H))


def _tables_kernel(x_ref, W_ref, o_ref):
    o_ref[0] = _dot(x_ref[...], W_ref[0])


def _tables(x, Wstack):
    S = Wstack.shape[0]
    out = pl.pallas_call(
        _tables_kernel,
        grid=(S, N // BN),
        in_specs=[
            pl.BlockSpec((BN, H), lambda s, k: (k, 0)),
            pl.BlockSpec((1, H, H), lambda s, k: (s, 0, 0)),
        ],
        out_specs=pl.BlockSpec((1, BN, H), lambda s, k: (s, k, 0)),
        out_shape=jax.ShapeDtypeStruct((S, N, H), jnp.float32),
    )(x, Wstack)
    return out.reshape(S * N, H)


def _mker_kernel(gd_ref, gs_ref, ea_ref, W3p_ref, bf_ref, m_ref, m2_ref):
    m = (gd_ref[...] + gs_ref[...]
         + _dot(ea_ref[...], W3p_ref[...]) + bf_ref[...])
    m_ref[...] = m
    m2_ref[...] = m * m


def _m_edges(g, ea, W3p, bf):
    KB = E // BE
    blk = pl.BlockSpec((BE, H), lambda i: (i, 0))
    return pl.pallas_call(
        _mker_kernel,
        grid=(KB,),
        in_specs=[
            blk,
            pl.BlockSpec((BE, H), lambda i: (i + KB, 0)),
            blk,
            pl.BlockSpec((H, H), lambda i: (0, 0)),
            pl.BlockSpec((1, H), lambda i: (0, 0)),
        ],
        out_specs=[blk, blk],
        out_shape=[jax.ShapeDtypeStruct((E, H), jnp.float32),
                   jax.ShapeDtypeStruct((E, H), jnp.float32)],
    )(g, g, ea, W3p, bf)


_NC = 4


def _minmax_kernel(m_ref, idx_ref, mm_ref):
    @pl.when(pl.program_id(0) == 0)
    def _init():
        mm_ref[...] = jnp.full((_NC, N, 2 * H), -jnp.inf, jnp.float32)

    def body(jj, carry):
        j = jj * _NC
        for cpy in range(_NC):
            d = idx_ref[0, 0, j + cpy]
            row = m_ref[pl.ds(j + cpy, 1), :]
            r2 = jnp.concatenate([row, -row], axis=1)
            cur = mm_ref[cpy, pl.ds(d, 1), :]
            mm_ref[cpy, pl.ds(d, 1), :] = jnp.maximum(cur, r2)
        return carry

    lax.fori_loop(0, BS // _NC, body, 0)


def _minmax(m, dst3):
    return pl.pallas_call(
        _minmax_kernel,
        grid=(E // BS,),
        in_specs=[
            pl.BlockSpec((BS, H), lambda i: (i, 0)),
            pl.BlockSpec((1, 1, BS), lambda i: (i, 0, 0), memory_space=pltpu.SMEM),
        ],
        out_specs=pl.BlockSpec((_NC, N, 2 * H), lambda i: (0, 0, 0)),
        out_shape=jax.ShapeDtypeStruct((_NC, N, 2 * H), jnp.float32),
    )(m, dst3)


def _nodeA_kernel(sa_ref, sb_ref, qa_ref, qb_ref, ca_ref, cb_ref, mm_ref,
                  x_ref, PW_ref, pb_ref, lin_ref, lb_ref,
                  out_ref, cs_ref, cq_ref):
    s = sa_ref[...] + sb_ref[...]
    s2 = qa_ref[...] + qb_ref[...]
    cnt = (ca_ref[...] + cb_ref[...])[:, 0:1]
    mm4 = mm_ref[...]
    mm = jnp.maximum(jnp.maximum(mm4[0], mm4[1]),
                     jnp.maximum(mm4[2], mm4[3]))
    deg = jnp.maximum(cnt, 1.0)
    has = cnt > 0.0
    mean = s / deg
    std = jnp.sqrt(jax.nn.relu(s2 / deg - mean * mean) + 1e-5)
    mn = jnp.where(has, -mm[:, H:], 0.0)
    mx = jnp.where(has, mm[:, :H], 0.0)
    logd = jnp.log(deg + 1.0)
    amp = logd * (1.0 / AVG_LOG)
    att = AVG_LOG / logd
    A = (mean, mn, mx, std)
    t0 = _dot(x_ref[...], PW_ref[0])
    t1 = sum(_dot(A[k], PW_ref[1 + k]) for k in range(4))
    t2 = sum(_dot(A[k], PW_ref[5 + k]) for k in range(4))
    t3 = sum(_dot(A[k], PW_ref[9 + k]) for k in range(4))
    out = t0 + t1 + amp * t2 + att * t3 + pb_ref[...]
    out = _dot(out, lin_ref[...]) + lb_ref[...]
    out_ref[...] = out

    @pl.when(pl.program_id(0) == 0)
    def _init():
        cs_ref[...] = jnp.zeros((1, H), jnp.float32)
        cq_ref[...] = jnp.zeros((1, H), jnp.float32)

    cs_ref[...] = cs_ref[...] + jnp.sum(out, axis=0, keepdims=True)
    cq_ref[...] = cq_ref[...] + jnp.sum(out * out, axis=0, keepdims=True)


def _nodeA(sp, qp, cp, mm, x, PW, pb, lin, lb):
    KN = N // BN
    blk = pl.BlockSpec((BN, H), lambda i: (i, 0))
    blk1 = pl.BlockSpec((BN, H), lambda i: (i + KN, 0))
    one = pl.BlockSpec((1, H), lambda i: (0, 0))
    return pl.pallas_call(
        _nodeA_kernel,
        grid=(KN,),
        in_specs=[blk, blk1, blk, blk1, blk, blk1,
                  pl.BlockSpec((_NC, BN, 2 * H), lambda i: (0, i, 0)),
                  blk,
                  pl.BlockSpec((13, H, H), lambda i: (0, 0, 0)),
                  one,
                  pl.BlockSpec((H, H), lambda i: (0, 0)),
                  one],
        out_specs=[blk, one, one],
        out_shape=[
            jax.ShapeDtypeStruct((N, H), jnp.float32),
            jax.ShapeDtypeStruct((1, H), jnp.float32),
            jax.ShapeDtypeStruct((1, H), jnp.float32),
        ],
    )(sp, sp, qp, qp, cp, cp, mm, x, PW, pb.reshape(1, H), lin,
      lb.reshape(1, H))


def _nodeB_kernel(out_ref, cs_ref, cq_ref, x_ref, bw_ref, bb_ref, xn_ref):
    mu = cs_ref[...] * (1.0 / N)
    var = cq_ref[...] * (1.0 / N) - mu * mu
    inv = jax.lax.rsqrt(var + 1e-5)
    bn = (out_ref[...] - mu) * inv * bw_ref[...] + bb_ref[...]
    xn_ref[...] = (x_ref[...] + jax.nn.relu(bn)) * 0.5


def _nodeB(out, cs, cq, x, bw, bb):
    blk = pl.BlockSpec((BN, H), lambda i: (i, 0))
    one = pl.BlockSpec((1, H), lambda i: (0, 0))
    return pl.pallas_call(
        _nodeB_kernel,
        grid=(N // BN,),
        in_specs=[blk, one, one, blk, one, one],
        out_specs=blk,
        out_shape=jax.ShapeDtypeStruct((N, H), jnp.float32),
    )(out, cs, cq, x, bw.reshape(1, H), bb.reshape(1, H))


def _em_kernel(ga_ref, gb_ref, ea_ref, B3_ref, b1_ref, W2_ref, b2_ref, o_ref):
    hid = (ga_ref[...] + gb_ref[...]
           + _dot(ea_ref[...], B3_ref[...]) + b1_ref[...])
    em = _dot(jax.nn.relu(hid), W2_ref[...]) + b2_ref[...]
    o_ref[...] = ea_ref[...] + em * 0.5


def _em_edges(g, off_a, off_b, ea, B3, b1, W2, b2):
    KB = E // BE
    return pl.pallas_call(
        _em_kernel,
        grid=(KB,),
        in_specs=[
            pl.BlockSpec((BE, H), lambda i, o=off_a: (i + o * KB, 0)),
            pl.BlockSpec((BE, H), lambda i, o=off_b: (i + o * KB, 0)),
            pl.BlockSpec((BE, H), lambda i: (i, 0)),
            pl.BlockSpec((H, H), lambda i: (0, 0)),
            pl.BlockSpec((1, H), lambda i: (0, 0)),
            pl.BlockSpec((H, H), lambda i: (0, 0)),
            pl.BlockSpec((1, H), lambda i: (0, 0)),
        ],
        out_specs=pl.BlockSpec((BE, H), lambda i: (i, 0)),
        out_shape=jax.ShapeDtypeStruct((E, H), jnp.float32),
    )(g, g, ea, B3, b1.reshape(1, H), W2, b2.reshape(1, H))


def kernel(x, edge_index, edge_attr, node_W, node_b, edge_W, edge_b, ee_W,
           ee_b, pre_W, pre_b, post_W, post_b, lin_W, lin_b, em1_W, em1_b,
           em2_W, em2_b, bn_w, bn_b):
    src = edge_index[0]
    dst = edge_index[1]
    dst3 = dst.reshape(E // BS, 1, BS)

    W3p, bf = _prep(ee_W, ee_b, pre_W, pre_b)

    xc = _enc_nodes(x, node_W, node_b)
    ea = _enc_edges(edge_attr, edge_W, edge_b)

    W1 = [pre_W[i][:H] for i in range(2)]
    W2 = [pre_W[i][H:2 * H] for i in range(2)]
    B1 = [em1_W[i][:H] for i in range(2)]
    B2 = [em1_W[i][H:2 * H] for i in range(2)]
    B3 = [em1_W[i][2 * H:] for i in range(2)]
    PW = [post_W[i].reshape(13, H, H) for i in range(2)]

    idx_ds = jnp.concatenate([dst, src + N])
    idx_emn = jnp.concatenate([src, dst + N, dst + 2 * N, src + 3 * N])
    idx_em = jnp.concatenate([src, dst + N])

    cp = _scatter_add(jnp.ones((E, H), jnp.float32), dst)
    for i in range(2):
        if i == 0:
            T = _tables(xc, jnp.stack([W1[0], W2[0]]))
            g = _gather(T, idx_ds)
            m, m2 = _m_edges(g, ea, W3p[i], bf[i])
        sp = _scatter_add(m, dst)
        qp = _scatter_add(m2, dst)
        mm = _minmax(m, dst3)
        out, cs, cq = _nodeA(sp, qp, cp, mm, xc, PW[i], post_b[i],
                             lin_W[i], lin_b[i])
        xc = _nodeB(out, cs, cq, xc, bn_w[i], bn_b[i])
        if i == 0:
            T = _tables(xc, jnp.stack([B1[0], B2[0], W1[1], W2[1]]))
            g = _gather(T, idx_emn)
            ea = _em_edges(g, 0, 1, ea, B3[0], em1_b[0], em2_W[0], em2_b[0])
            m, m2 = _m_edges(g[2 * E:], ea, W3p[1], bf[1])
        else:
            T = _tables(xc, jnp.stack([B1[1], B2[1]]))
            g = _gather(T, idx_em)
            ea = _em_edges(g, 0, 1, ea, B3[1], em1_b[1], em2_W[1], em2_b[1])
    return xc, ea


# speedup vs baseline: 1.5328x; 1.0350x over previous
import functools
import math

import jax
import jax.numpy as jnp
from jax import lax
from jax.experimental import pallas as pl
from jax.experimental.pallas import tpu as pltpu
from jax.experimental.pallas import tpu_sc as plsc

N = 10000
E = 320000
H = 128
AVG_LOG = math.log(33.0)

BE = 1600
BN = 1000
BS = 2000


_SC_NC = 2
_SC_NS = 16
_NW = _SC_NC * _SC_NS


def _maybe_when(cond, fn):
    if isinstance(cond, bool):
        if cond:
            fn()
    else:
        pl.when(cond)(fn)


@functools.lru_cache(maxsize=None)
def _sc_gather(Mrows, Q):
    qpw = Q // _NW
    CH = 400
    assert qpw % CH == 0 and Q % _NW == 0
    mesh = plsc.VectorSubcoreMesh(core_axis_name="c", subcore_axis_name="s")

    @functools.partial(
        pl.kernel,
        mesh=mesh,
        out_type=jax.ShapeDtypeStruct((Q, H), jnp.float32),
        scratch_types=[
            pltpu.VMEM((CH,), jnp.int32),
            pltpu.VMEM((CH,), jnp.int32),
            pltpu.VMEM((CH, H), jnp.float32),
            pltpu.VMEM((CH, H), jnp.float32),
        ] + [pltpu.SemaphoreType.DMA] * 6,
    )
    def gk(table_hbm, idx_hbm, out_hbm, idx_v0, idx_v1, rows_v0, rows_v1,
           isem0, isem1, gsem0, gsem1, osem0, osem1):
        wid = lax.axis_index("s") * _SC_NC + lax.axis_index("c")
        w0 = pl.multiple_of(wid * qpw, 8)
        K = qpw // CH
        idx_v = (idx_v0, idx_v1)
        rows_v = (rows_v0, rows_v1)
        isem = (isem0, isem1)
        gsem = (gsem0, gsem1)
        osem = (osem0, osem1)

        def hslice(k):
            return idx_hbm.at[pl.ds(pl.multiple_of(w0 + k * CH, 8), CH)]

        def oslice(k):
            return out_hbm.at[pl.ds(pl.multiple_of(w0 + k * CH, 8), CH)]

        def idx_cp(k, b):
            return pltpu.make_async_copy(hslice(k), idx_v[b], isem[b])

        def gat_cp(k, b):
            return pltpu.make_async_copy(table_hbm.at[idx_v[b]], rows_v[b],
                                         gsem[b])

        def out_cp(k, b):
            return pltpu.make_async_copy(rows_v[b], oslice(k), osem[b])

        idx_cp(0, 0).start()

        def phase(k, b, nb):
            idx_cp(k, b).wait()

            @pl.when(k >= 1)
            def _():
                gat_cp(k - 1, nb).wait()
                out_cp(k - 1, nb).start()

            @pl.when(k + 1 < K)
            def _():
                idx_cp(k + 1, nb).start()

            @pl.when(k >= 2)
            def _():
                out_cp(k - 2, b).wait()

            gat_cp(k, b).start()

        def body(g, carry):
            phase(2 * g, 0, 1)
            phase(2 * g + 1, 1, 0)
            return carry

        lax.fori_loop(0, K // 2, body, 0)
        gat_cp(K - 1, (K - 1) % 2).wait()
        out_cp(K - 2, (K - 2) % 2).wait()
        pltpu.sync_copy(rows_v[(K - 1) % 2], oslice(K - 1))

    return gk


def _gather(table, idx):
    return _sc_gather(table.shape[0], idx.shape[0])(table, idx)


_SCH = 80
_NP = 10240
_ROWS_T = _NP // _SC_NS


@functools.lru_cache(maxsize=None)
def _sc_scatter_add(D):
    epc = E // _NW
    mesh = plsc.VectorSubcoreMesh(core_axis_name="c", subcore_axis_name="s")
    zrows = 128

    @functools.partial(
        pl.kernel,
        mesh=mesh,
        out_type=jax.ShapeDtypeStruct((2 * _NP, D), jnp.float32),
        scratch_types=[
            pltpu.VMEM_SHARED((_NP, D), jnp.float32),
            pltpu.VMEM((_SCH,), jnp.int32),
            pltpu.VMEM((_SCH,), jnp.int32),
            pltpu.VMEM((_SCH, D), jnp.float32),
            pltpu.VMEM((_SCH, D), jnp.float32),
            pltpu.VMEM((zrows, D), jnp.float32),
        ] + [pltpu.SemaphoreType.DMA] * 6,
    )
    def sk(*refs):
        (vals_hbm, idx_hbm, out_hbm, acc, idx_v0, idx_v1, vals_v0, vals_v1,
         zero_v, isem0, isem1, vsem0, vsem1, ssem0, ssem1) = refs
        c = lax.axis_index("c")
        s = lax.axis_index("s")
        idx_v = (idx_v0, idx_v1)
        vals_v = (vals_v0, vals_v1)
        isem = (isem0, isem1)
        vsem = (vsem0, vsem1)
        ssem = (ssem0, ssem1)

        zbase = s * _ROWS_T
        for zc in range(_ROWS_T // zrows):
            zero_v[...] = jnp.zeros((zrows, D), jnp.float32)
            pltpu.sync_copy(zero_v, acc.at[pl.ds(zbase + zc * zrows, zrows)])
        plsc.subcore_barrier()

        ebase = c * (E // 2) + s * epc
        K = epc // _SCH

        def eslice(ref, k):
            return ref.at[pl.ds(pl.multiple_of(ebase + k * _SCH, 8), _SCH)]

        def ld_cp(k, b):
            return (pltpu.make_async_copy(eslice(idx_hbm, k), idx_v[b], isem[b]),
                    pltpu.make_async_copy(eslice(vals_hbm, k), vals_v[b], vsem[b]))

        def sc_start(b):
            pltpu.async_copy(vals_v[b], acc.at[idx_v[b]], ssem[b], add=True)

        def sc_wait(b):
            pltpu.make_async_copy(vals_v[b], acc.at[idx_v[b]], ssem[b]).wait()

        for cp in ld_cp(0, 0):
            cp.start()

        def phase(k, b, nb):
            for cp in ld_cp(k, b):
                cp.wait()

            _maybe_when(k >= 1, lambda: sc_wait(nb))
            sc_start(b)

            def _start_next():
                for cp in ld_cp(k + 1, nb):
                    cp.start()

            _maybe_when(k + 1 < K, _start_next)

        def body(g, carry):
            phase(2 * g, 0, 1)
            phase(2 * g + 1, 1, 0)
            return carry

        lax.fori_loop(0, K // 2, body, 0)
        phase(K - 1, (K - 1) % 2, K % 2)
        sc_wait((K - 1) % 2)
        plsc.subcore_barrier()
        obase = c * _NP + s * _ROWS_T
        pltpu.sync_copy(acc.at[pl.ds(s * _ROWS_T, _ROWS_T)],
                        out_hbm.at[pl.ds(obase, _ROWS_T)])

    return sk


def _scatter_add(vals, idx):
    out = _sc_scatter_add(vals.shape[1])(vals, idx)
    return jnp.concatenate([out[:N], out[_NP:_NP + N]], axis=0)


def _dot(a, b):
    return jnp.dot(a, b, preferred_element_type=jnp.float32)


def _prep_kernel(eeW_ref, eeb_ref, preW_ref, preb_ref, W3p_ref, bf_ref):
    for i in range(2):
        W3 = preW_ref[i][2 * H:, :]
        W3p_ref[i] = _dot(eeW_ref[i], W3)
        bf_ref[i] = preb_ref[i] + _dot(eeb_ref[i], W3)


def _prep(ee_W, ee_b, pre_W, pre_b):
    return pl.pallas_call(
        _prep_kernel,
        out_shape=(
            jax.ShapeDtypeStruct((2, H, H), jnp.float32),
            jax.ShapeDtypeStruct((2, 1, H), jnp.float32),
        ),
    )(ee_W, ee_b.reshape(2, 1, H), pre_W, pre_b.reshape(2, 1, H))


def _encN_kernel(x_ref, W_ref, b_ref, o_ref):
    o_ref[...] = _dot(x_ref[...], W_ref[...]) + b_ref[...]


def _enc_nodes(x, W, b):
    return pl.pallas_call(
        _encN_kernel,
        grid=(N // BN,),
        in_specs=[
            pl.BlockSpec((BN, H), lambda i: (i, 0)),
            pl.BlockSpec((H, H), lambda i: (0, 0)),
            pl.BlockSpec((1, H), lambda i: (0, 0)),
        ],
        out_specs=pl.BlockSpec((BN, H), lambda i: (i, 0)),
        out_shape=jax.ShapeDtypeStruct((N, H), jnp.float32),
    )(x, W, b.reshape(1, H))


def _enc_edges(ea, W, b):
    D = ea.shape[1]
    return pl.pallas_call(
        _encN_kernel,
        grid=(E // BE,),
        in_specs=[
            pl.BlockSpec((BE, D), lambda i: (i, 0)),
            pl.BlockSpec((D, H), lambda i: (0, 0)),
            pl.BlockSpec((1, H), lambda i: (0, 0)),
        ],
        out_specs=pl.BlockSpec((BE, H), lambda i: (i, 0)),
        out_shape=jax.ShapeDtypeStruct((E, H), jnp.float32),
    )(ea, W, b.reshape(1, H))


def _tables_kernel(x_ref, W_ref, o_ref):
    o_ref[0] = _dot(x_ref[...], W_ref[0])


def _tables(x, Wstack):
    S = Wstack.shape[0]
    out = pl.pallas_call(
        _tables_kernel,
        grid=(S, N // BN),
        in_specs=[
            pl.BlockSpec((BN, H), lambda s, k: (k, 0)),
            pl.BlockSpec((1, H, H), lambda s, k: (s, 0, 0)),
        ],
        out_specs=pl.BlockSpec((1, BN, H), lambda s, k: (s, k, 0)),
        out_shape=jax.ShapeDtypeStruct((S, N, H), jnp.float32),
    )(x, Wstack)
    return out.reshape(S * N, H)


def _mker_kernel(gd_ref, gs_ref, ea_ref, W3p_ref, bf_ref, m_ref, m2_ref):
    m = (gd_ref[...] + gs_ref[...]
         + _dot(ea_ref[...], W3p_ref[...]) + bf_ref[...])
    m_ref[...] = m
    m2_ref[...] = m * m


def _m_edges(g, ea, W3p, bf):
    KB = E // BE
    blk = pl.BlockSpec((BE, H), lambda i: (i, 0))
    return pl.pallas_call(
        _mker_kernel,
        grid=(KB,),
        in_specs=[
            blk,
            pl.BlockSpec((BE, H), lambda i: (i + KB, 0)),
            blk,
            pl.BlockSpec((H, H), lambda i: (0, 0)),
            pl.BlockSpec((1, H), lambda i: (0, 0)),
        ],
        out_specs=[blk, blk],
        out_shape=[jax.ShapeDtypeStruct((E, H), jnp.float32),
                   jax.ShapeDtypeStruct((E, H), jnp.float32)],
    )(g, g, ea, W3p, bf)


_NC = 4


def _minmax_kernel(m_ref, idx_ref, mm_ref):
    @pl.when(pl.program_id(0) == 0)
    def _init():
        mm_ref[...] = jnp.full((_NC, N, 2 * H), -jnp.inf, jnp.float32)

    def body(jj, carry):
        j = jj * _NC
        for cpy in range(_NC):
            d = idx_ref[0, 0, j + cpy]
            row = m_ref[pl.ds(j + cpy, 1), :]
            r2 = jnp.concatenate([row, -row], axis=1)
            cur = mm_ref[cpy, pl.ds(d, 1), :]
            mm_ref[cpy, pl.ds(d, 1), :] = jnp.maximum(cur, r2)
        return carry

    lax.fori_loop(0, BS // _NC, body, 0)


def _minmax(m, dst3):
    return pl.pallas_call(
        _minmax_kernel,
        grid=(E // BS,),
        in_specs=[
            pl.BlockSpec((BS, H), lambda i: (i, 0)),
            pl.BlockSpec((1, 1, BS), lambda i: (i, 0, 0), memory_space=pltpu.SMEM),
        ],
        out_specs=pl.BlockSpec((_NC, N, 2 * H), lambda i: (0, 0, 0)),
        out_shape=jax.ShapeDtypeStruct((_NC, N, 2 * H), jnp.float32),
    )(m, dst3)


def _nodeA_kernel(sa_ref, sb_ref, qa_ref, qb_ref, ca_ref, cb_ref, mm_ref,
                  x_ref, PW_ref, pb_ref, lin_ref, lb_ref,
                  out_ref, cs_ref, cq_ref):
    s = sa_ref[...] + sb_ref[...]
    s2 = qa_ref[...] + qb_ref[...]
    cnt = (ca_ref[...] + cb_ref[...])[:, 0:1]
    mm4 = mm_ref[...]
    mm = jnp.maximum(jnp.maximum(mm4[0], mm4[1]),
                     jnp.maximum(mm4[2], mm4[3]))
    deg = jnp.maximum(cnt, 1.0)
    has = cnt > 0.0
    mean = s / deg
    std = jnp.sqrt(jax.nn.relu(s2 / deg - mean * mean) + 1e-5)
    mn = jnp.where(has, -mm[:, H:], 0.0)
    mx = jnp.where(has, mm[:, :H], 0.0)
    logd = jnp.log(deg + 1.0)
    amp = logd * (1.0 / AVG_LOG)
    att = AVG_LOG / logd
    A = (mean, mn, mx, std)
    t0 = _dot(x_ref[...], PW_ref[0])
    t1 = sum(_dot(A[k], PW_ref[1 + k]) for k in range(4))
    t2 = sum(_dot(A[k], PW_ref[5 + k]) for k in range(4))
    t3 = sum(_dot(A[k], PW_ref[9 + k]) for k in range(4))
    out = t0 + t1 + amp * t2 + att * t3 + pb_ref[...]
    out = _dot(out, lin_ref[...]) + lb_ref[...]
    out_ref[...] = out

    @pl.when(pl.program_id(0) == 0)
    def _init():
        cs_ref[...] = jnp.zeros((1, H), jnp.float32)
        cq_ref[...] = jnp.zeros((1, H), jnp.float32)

    cs_ref[...] = cs_ref[...] + jnp.sum(out, axis=0, keepdims=True)
    cq_ref[...] = cq_ref[...] + jnp.sum(out * out, axis=0, keepdims=True)


def _nodeA(sp, qp, cp, mm, x, PW, pb, lin, lb):
    KN = N // BN
    blk = pl.BlockSpec((BN, H), lambda i: (i, 0))
    blk1 = pl.BlockSpec((BN, H), lambda i: (i + KN, 0))
    one = pl.BlockSpec((1, H), lambda i: (0, 0))
    return pl.pallas_call(
        _nodeA_kernel,
        grid=(KN,),
        in_specs=[blk, blk1, blk, blk1, blk, blk1,
                  pl.BlockSpec((_NC, BN, 2 * H), lambda i: (0, i, 0)),
                  blk,
                  pl.BlockSpec((13, H, H), lambda i: (0, 0, 0)),
                  one,
                  pl.BlockSpec((H, H), lambda i: (0, 0)),
                  one],
        out_specs=[blk, one, one],
        out_shape=[
            jax.ShapeDtypeStruct((N, H), jnp.float32),
            jax.ShapeDtypeStruct((1, H), jnp.float32),
            jax.ShapeDtypeStruct((1, H), jnp.float32),
        ],
    )(sp, sp, qp, qp, cp, cp, mm, x, PW, pb.reshape(1, H), lin,
      lb.reshape(1, H))


def _nodeB_kernel(out_ref, cs_ref, cq_ref, x_ref, bw_ref, bb_ref, xn_ref):
    mu = cs_ref[...] * (1.0 / N)
    var = cq_ref[...] * (1.0 / N) - mu * mu
    inv = jax.lax.rsqrt(var + 1e-5)
    bn = (out_ref[...] - mu) * inv * bw_ref[...] + bb_ref[...]
    xn_ref[...] = (x_ref[...] + jax.nn.relu(bn)) * 0.5


def _nodeB(out, cs, cq, x, bw, bb):
    blk = pl.BlockSpec((BN, H), lambda i: (i, 0))
    one = pl.BlockSpec((1, H), lambda i: (0, 0))
    return pl.pallas_call(
        _nodeB_kernel,
        grid=(N // BN,),
        in_specs=[blk, one, one, blk, one, one],
        out_specs=blk,
        out_shape=jax.ShapeDtypeStruct((N, H), jnp.float32),
    )(out, cs, cq, x, bw.reshape(1, H), bb.reshape(1, H))


def _em_kernel(ga_ref, gb_ref, ea_ref, B3_ref, b1_ref, W2_ref, b2_ref, o_ref):
    hid = (ga_ref[...] + gb_ref[...]
           + _dot(ea_ref[...], B3_ref[...]) + b1_ref[...])
    em = _dot(jax.nn.relu(hid), W2_ref[...]) + b2_ref[...]
    o_ref[...] = ea_ref[...] + em * 0.5


def _em_edges(g, off_a, off_b, ea, B3, b1, W2, b2):
    KB = E // BE
    return pl.pallas_call(
        _em_kernel,
        grid=(KB,),
        in_specs=[
            pl.BlockSpec((BE, H), lambda i, o=off_a: (i + o * KB, 0)),
            pl.BlockSpec((BE, H), lambda i, o=off_b: (i + o * KB, 0)),
            pl.BlockSpec((BE, H), lambda i: (i, 0)),
            pl.BlockSpec((H, H), lambda i: (0, 0)),
            pl.BlockSpec((1, H), lambda i: (0, 0)),
            pl.BlockSpec((H, H), lambda i: (0, 0)),
            pl.BlockSpec((1, H), lambda i: (0, 0)),
        ],
        out_specs=pl.BlockSpec((BE, H), lambda i: (i, 0)),
        out_shape=jax.ShapeDtypeStruct((E, H), jnp.float32),
    )(g, g, ea, B3, b1.reshape(1, H), W2, b2.reshape(1, H))


def kernel(x, edge_index, edge_attr, node_W, node_b, edge_W, edge_b, ee_W,
           ee_b, pre_W, pre_b, post_W, post_b, lin_W, lin_b, em1_W, em1_b,
           em2_W, em2_b, bn_w, bn_b):
    src = edge_index[0]
    dst = edge_index[1]
    dst3 = dst.reshape(E // BS, 1, BS)

    W3p, bf = _prep(ee_W, ee_b, pre_W, pre_b)

    xc = _enc_nodes(x, node_W, node_b)
    ea = _enc_edges(edge_attr, edge_W, edge_b)

    W1 = [pre_W[i][:H] for i in range(2)]
    W2 = [pre_W[i][H:2 * H] for i in range(2)]
    B1 = [em1_W[i][:H] for i in range(2)]
    B2 = [em1_W[i][H:2 * H] for i in range(2)]
    B3 = [em1_W[i][2 * H:] for i in range(2)]
    PW = [post_W[i].reshape(13, H, H) for i in range(2)]

    idx_ds = jnp.concatenate([dst, src + N])
    idx_emn = jnp.concatenate([src, dst + N, dst + 2 * N, src + 3 * N])
    idx_em = jnp.concatenate([src, dst + N])

    cp = _scatter_add(jnp.ones((E, H), jnp.float32), dst)
    for i in range(2):
        if i == 0:
            T = _tables(xc, jnp.stack([W1[0], W2[0]]))
            g = _gather(T, idx_ds)
            m, m2 = _m_edges(g, ea, W3p[i], bf[i])
        sp = _scatter_add(m, dst)
        qp = _scatter_add(m2, dst)
        mm = _minmax(m, dst3)
        out, cs, cq = _nodeA(sp, qp, cp, mm, xc, PW[i], post_b[i],
                             lin_W[i], lin_b[i])
        xc = _nodeB(out, cs, cq, xc, bn_w[i], bn_b[i])
        if i == 0:
            T = _tables(xc, jnp.stack([B1[0], B2[0], W1[1], W2[1]]))
            g = _gather(T, idx_emn)
            ea = _em_edges(g, 0, 1, ea, B3[0], em1_b[0], em2_W[0], em2_b[0])
            m, m2 = _m_edges(g[2 * E:], ea, W3p[1], bf[1])
        else:
            T = _tables(xc, jnp.stack([B1[1], B2[1]]))
            g = _gather(T, idx_em)
            ea = _em_edges(g, 0, 1, ea, B3[1], em1_b[1], em2_W[1], em2_b[1])
    return xc, ea


# speedup vs baseline: 1.5947x; 1.0404x over previous
import functools
import math

import jax
import jax.numpy as jnp
from jax import lax
from jax.experimental import pallas as pl
from jax.experimental.pallas import tpu as pltpu
from jax.experimental.pallas import tpu_sc as plsc

N = 10000
E = 320000
H = 128
AVG_LOG = math.log(33.0)

BE = 1600
BN = 1000
BS = 2000


_SC_NC = 2
_SC_NS = 16
_NW = _SC_NC * _SC_NS


def _maybe_when(cond, fn):
    if isinstance(cond, bool):
        if cond:
            fn()
    else:
        pl.when(cond)(fn)


@functools.lru_cache(maxsize=None)
def _sc_gather(Mrows, Q):
    qpw = Q // _NW
    CH = 400
    assert qpw % CH == 0 and Q % _NW == 0
    mesh = plsc.VectorSubcoreMesh(core_axis_name="c", subcore_axis_name="s")

    @functools.partial(
        pl.kernel,
        mesh=mesh,
        out_type=jax.ShapeDtypeStruct((Q, H), jnp.float32),
        scratch_types=[
            pltpu.VMEM((CH,), jnp.int32),
            pltpu.VMEM((CH,), jnp.int32),
            pltpu.VMEM((CH, H), jnp.float32),
            pltpu.VMEM((CH, H), jnp.float32),
        ] + [pltpu.SemaphoreType.DMA] * 6,
    )
    def gk(table_hbm, idx_hbm, out_hbm, idx_v0, idx_v1, rows_v0, rows_v1,
           isem0, isem1, gsem0, gsem1, osem0, osem1):
        wid = lax.axis_index("s") * _SC_NC + lax.axis_index("c")
        w0 = pl.multiple_of(wid * qpw, 8)
        K = qpw // CH
        idx_v = (idx_v0, idx_v1)
        rows_v = (rows_v0, rows_v1)
        isem = (isem0, isem1)
        gsem = (gsem0, gsem1)
        osem = (osem0, osem1)

        def hslice(k):
            return idx_hbm.at[pl.ds(pl.multiple_of(w0 + k * CH, 8), CH)]

        def oslice(k):
            return out_hbm.at[pl.ds(pl.multiple_of(w0 + k * CH, 8), CH)]

        def idx_cp(k, b):
            return pltpu.make_async_copy(hslice(k), idx_v[b], isem[b])

        def gat_cp(k, b):
            return pltpu.make_async_copy(table_hbm.at[idx_v[b]], rows_v[b],
                                         gsem[b])

        def out_cp(k, b):
            return pltpu.make_async_copy(rows_v[b], oslice(k), osem[b])

        idx_cp(0, 0).start()

        def phase(k, b, nb):
            idx_cp(k, b).wait()

            @pl.when(k >= 1)
            def _():
                gat_cp(k - 1, nb).wait()
                out_cp(k - 1, nb).start()

            @pl.when(k + 1 < K)
            def _():
                idx_cp(k + 1, nb).start()

            @pl.when(k >= 2)
            def _():
                out_cp(k - 2, b).wait()

            gat_cp(k, b).start()

        def body(g, carry):
            phase(2 * g, 0, 1)
            phase(2 * g + 1, 1, 0)
            return carry

        lax.fori_loop(0, K // 2, body, 0)
        gat_cp(K - 1, (K - 1) % 2).wait()
        out_cp(K - 2, (K - 2) % 2).wait()
        pltpu.sync_copy(rows_v[(K - 1) % 2], oslice(K - 1))

    return gk


def _gather(table, idx):
    return _sc_gather(table.shape[0], idx.shape[0])(table, idx)


_SCH = 80
_NP = 10240
_ROWS_T = _NP // _SC_NS


@functools.lru_cache(maxsize=None)
def _sc_scatter_add(D):
    epc = E // _NW
    mesh = plsc.VectorSubcoreMesh(core_axis_name="c", subcore_axis_name="s")
    zrows = 128

    @functools.partial(
        pl.kernel,
        mesh=mesh,
        out_type=jax.ShapeDtypeStruct((2 * _NP, D), jnp.float32),
        scratch_types=[
            pltpu.VMEM_SHARED((_NP, D), jnp.float32),
            pltpu.VMEM((_SCH,), jnp.int32),
            pltpu.VMEM((_SCH,), jnp.int32),
            pltpu.VMEM((_SCH, D), jnp.float32),
            pltpu.VMEM((_SCH, D), jnp.float32),
            pltpu.VMEM((zrows, D), jnp.float32),
        ] + [pltpu.SemaphoreType.DMA] * 6,
    )
    def sk(*refs):
        (vals_hbm, idx_hbm, out_hbm, acc, idx_v0, idx_v1, vals_v0, vals_v1,
         zero_v, isem0, isem1, vsem0, vsem1, ssem0, ssem1) = refs
        c = lax.axis_index("c")
        s = lax.axis_index("s")
        idx_v = (idx_v0, idx_v1)
        vals_v = (vals_v0, vals_v1)
        isem = (isem0, isem1)
        vsem = (vsem0, vsem1)
        ssem = (ssem0, ssem1)

        zbase = s * _ROWS_T
        for zc in range(_ROWS_T // zrows):
            zero_v[...] = jnp.zeros((zrows, D), jnp.float32)
            pltpu.sync_copy(zero_v, acc.at[pl.ds(zbase + zc * zrows, zrows)])
        plsc.subcore_barrier()

        ebase = c * (E // 2) + s * epc
        K = epc // _SCH

        def eslice(ref, k):
            return ref.at[pl.ds(pl.multiple_of(ebase + k * _SCH, 8), _SCH)]

        def ld_cp(k, b):
            return (pltpu.make_async_copy(eslice(idx_hbm, k), idx_v[b], isem[b]),
                    pltpu.make_async_copy(eslice(vals_hbm, k), vals_v[b], vsem[b]))

        def sc_start(b):
            pltpu.async_copy(vals_v[b], acc.at[idx_v[b]], ssem[b], add=True)

        def sc_wait(b):
            pltpu.make_async_copy(vals_v[b], acc.at[idx_v[b]], ssem[b]).wait()

        for cp in ld_cp(0, 0):
            cp.start()

        def phase(k, b, nb):
            for cp in ld_cp(k, b):
                cp.wait()

            _maybe_when(k >= 1, lambda: sc_wait(nb))
            sc_start(b)

            def _start_next():
                for cp in ld_cp(k + 1, nb):
                    cp.start()

            _maybe_when(k + 1 < K, _start_next)

        def body(g, carry):
            phase(2 * g, 0, 1)
            phase(2 * g + 1, 1, 0)
            return carry

        lax.fori_loop(0, K // 2, body, 0)
        phase(K - 1, (K - 1) % 2, K % 2)
        sc_wait((K - 1) % 2)
        plsc.subcore_barrier()
        obase = c * _NP + s * _ROWS_T
        pltpu.sync_copy(acc.at[pl.ds(s * _ROWS_T, _ROWS_T)],
                        out_hbm.at[pl.ds(obase, _ROWS_T)])

    return sk


def _scatter_add(vals, idx):
    out = _sc_scatter_add(vals.shape[1])(vals, idx)
    return jnp.concatenate([out[:N], out[_NP:_NP + N]], axis=0)


def _dot(a, b):
    return jnp.dot(a, b, preferred_element_type=jnp.float32)


def _prep_kernel(eeW_ref, eeb_ref, preW_ref, preb_ref, W3p_ref, bf_ref):
    for i in range(2):
        W3 = preW_ref[i][2 * H:, :]
        W3p_ref[i] = _dot(eeW_ref[i], W3)
        bf_ref[i] = preb_ref[i] + _dot(eeb_ref[i], W3)


def _prep(ee_W, ee_b, pre_W, pre_b):
    return pl.pallas_call(
        _prep_kernel,
        out_shape=(
            jax.ShapeDtypeStruct((2, H, H), jnp.float32),
            jax.ShapeDtypeStruct((2, 1, H), jnp.float32),
        ),
    )(ee_W, ee_b.reshape(2, 1, H), pre_W, pre_b.reshape(2, 1, H))


def _encN_kernel(x_ref, W_ref, b_ref, o_ref):
    o_ref[...] = _dot(x_ref[...], W_ref[...]) + b_ref[...]


def _enc_nodes(x, W, b):
    return pl.pallas_call(
        _encN_kernel,
        grid=(N // BN,),
        in_specs=[
            pl.BlockSpec((BN, H), lambda i: (i, 0)),
            pl.BlockSpec((H, H), lambda i: (0, 0)),
            pl.BlockSpec((1, H), lambda i: (0, 0)),
        ],
        out_specs=pl.BlockSpec((BN, H), lambda i: (i, 0)),
        out_shape=jax.ShapeDtypeStruct((N, H), jnp.float32),
    )(x, W, b.reshape(1, H))


def _enc_edges(ea, W, b):
    D = ea.shape[1]
    return pl.pallas_call(
        _encN_kernel,
        grid=(E // BE,),
        in_specs=[
            pl.BlockSpec((BE, D), lambda i: (i, 0)),
            pl.BlockSpec((D, H), lambda i: (0, 0)),
            pl.BlockSpec((1, H), lambda i: (0, 0)),
        ],
        out_specs=pl.BlockSpec((BE, H), lambda i: (i, 0)),
        out_shape=jax.ShapeDtypeStruct((E, H), jnp.float32),
    )(ea, W, b.reshape(1, H))


def _tables_kernel(x_ref, W_ref, o_ref):
    o_ref[0] = _dot(x_ref[...], W_ref[0])


def _tables(x, Wstack):
    S = Wstack.shape[0]
    out = pl.pallas_call(
        _tables_kernel,
        grid=(S, N // BN),
        in_specs=[
            pl.BlockSpec((BN, H), lambda s, k: (k, 0)),
            pl.BlockSpec((1, H, H), lambda s, k: (s, 0, 0)),
        ],
        out_specs=pl.BlockSpec((1, BN, H), lambda s, k: (s, k, 0)),
        out_shape=jax.ShapeDtypeStruct((S, N, H), jnp.float32),
    )(x, Wstack)
    return out.reshape(S * N, H)


def _mker_kernel(gd_ref, gs_ref, ea_ref, W3p_ref, bf_ref, m_ref, m2_ref):
    m = (gd_ref[...] + gs_ref[...]
         + _dot(ea_ref[...], W3p_ref[...]) + bf_ref[...])
    m_ref[...] = m
    m2_ref[...] = m * m


def _m_edges(g, ea, W3p, bf):
    KB = E // BE
    blk = pl.BlockSpec((BE, H), lambda i: (i, 0))
    return pl.pallas_call(
        _mker_kernel,
        grid=(KB,),
        in_specs=[
            blk,
            pl.BlockSpec((BE, H), lambda i: (i + KB, 0)),
            blk,
            pl.BlockSpec((H, H), lambda i: (0, 0)),
            pl.BlockSpec((1, H), lambda i: (0, 0)),
        ],
        out_specs=[blk, blk],
        out_shape=[jax.ShapeDtypeStruct((E, H), jnp.float32),
                   jax.ShapeDtypeStruct((E, H), jnp.float32)],
    )(g, g, ea, W3p, bf)


_NC = 5


def _minmax_kernel(m_ref, idx_ref, mm_ref):
    @pl.when(pl.program_id(0) == 0)
    def _init():
        mm_ref[...] = jnp.full((_NC, N, 2 * H), -jnp.inf, jnp.float32)

    def body(jj, carry):
        j = jj * _NC
        for cpy in range(_NC):
            d = idx_ref[0, 0, j + cpy]
            row = m_ref[pl.ds(j + cpy, 1), :]
            r2 = jnp.concatenate([row, -row], axis=1)
            cur = mm_ref[cpy, pl.ds(d, 1), :]
            mm_ref[cpy, pl.ds(d, 1), :] = jnp.maximum(cur, r2)
        return carry

    lax.fori_loop(0, BS // _NC, body, 0)


def _minmax(m, dst3):
    return pl.pallas_call(
        _minmax_kernel,
        grid=(E // BS,),
        in_specs=[
            pl.BlockSpec((BS, H), lambda i: (i, 0)),
            pl.BlockSpec((1, 1, BS), lambda i: (i, 0, 0), memory_space=pltpu.SMEM),
        ],
        out_specs=pl.BlockSpec((_NC, N, 2 * H), lambda i: (0, 0, 0)),
        out_shape=jax.ShapeDtypeStruct((_NC, N, 2 * H), jnp.float32),
    )(m, dst3)


def _nodeA_kernel(sa_ref, sb_ref, qa_ref, qb_ref, ca_ref, cb_ref, mm_ref,
                  x_ref, PW_ref, pb_ref, lin_ref, lb_ref,
                  out_ref, cs_ref, cq_ref):
    s = sa_ref[...] + sb_ref[...]
    s2 = qa_ref[...] + qb_ref[...]
    cnt = (ca_ref[...] + cb_ref[...])[:, 0:1]
    mm4 = mm_ref[...]
    mm = mm4[0]
    for _c in range(1, _NC):
        mm = jnp.maximum(mm, mm4[_c])
    deg = jnp.maximum(cnt, 1.0)
    has = cnt > 0.0
    mean = s / deg
    std = jnp.sqrt(jax.nn.relu(s2 / deg - mean * mean) + 1e-5)
    mn = jnp.where(has, -mm[:, H:], 0.0)
    mx = jnp.where(has, mm[:, :H], 0.0)
    logd = jnp.log(deg + 1.0)
    amp = logd * (1.0 / AVG_LOG)
    att = AVG_LOG / logd
    A = (mean, mn, mx, std)
    t0 = _dot(x_ref[...], PW_ref[0])
    t1 = sum(_dot(A[k], PW_ref[1 + k]) for k in range(4))
    t2 = sum(_dot(A[k], PW_ref[5 + k]) for k in range(4))
    t3 = sum(_dot(A[k], PW_ref[9 + k]) for k in range(4))
    out = t0 + t1 + amp * t2 + att * t3 + pb_ref[...]
    out = _dot(out, lin_ref[...]) + lb_ref[...]
    out_ref[...] = out

    @pl.when(pl.program_id(0) == 0)
    def _init():
        cs_ref[...] = jnp.zeros((1, H), jnp.float32)
        cq_ref[...] = jnp.zeros((1, H), jnp.float32)

    cs_ref[...] = cs_ref[...] + jnp.sum(out, axis=0, keepdims=True)
    cq_ref[...] = cq_ref[...] + jnp.sum(out * out, axis=0, keepdims=True)


def _nodeA(sp, qp, cp, mm, x, PW, pb, lin, lb):
    KN = N // BN
    blk = pl.BlockSpec((BN, H), lambda i: (i, 0))
    blk1 = pl.BlockSpec((BN, H), lambda i: (i + KN, 0))
    one = pl.BlockSpec((1, H), lambda i: (0, 0))
    return pl.pallas_call(
        _nodeA_kernel,
        grid=(KN,),
        in_specs=[blk, blk1, blk, blk1, blk, blk1,
                  pl.BlockSpec((_NC, BN, 2 * H), lambda i: (0, i, 0)),
                  blk,
                  pl.BlockSpec((13, H, H), lambda i: (0, 0, 0)),
                  one,
                  pl.BlockSpec((H, H), lambda i: (0, 0)),
                  one],
        out_specs=[blk, one, one],
        out_shape=[
            jax.ShapeDtypeStruct((N, H), jnp.float32),
            jax.ShapeDtypeStruct((1, H), jnp.float32),
            jax.ShapeDtypeStruct((1, H), jnp.float32),
        ],
    )(sp, sp, qp, qp, cp, cp, mm, x, PW, pb.reshape(1, H), lin,
      lb.reshape(1, H))


def _nodeB_kernel(out_ref, cs_ref, cq_ref, x_ref, bw_ref, bb_ref, xn_ref):
    mu = cs_ref[...] * (1.0 / N)
    var = cq_ref[...] * (1.0 / N) - mu * mu
    inv = jax.lax.rsqrt(var + 1e-5)
    bn = (out_ref[...] - mu) * inv * bw_ref[...] + bb_ref[...]
    xn_ref[...] = (x_ref[...] + jax.nn.relu(bn)) * 0.5


def _nodeB(out, cs, cq, x, bw, bb):
    blk = pl.BlockSpec((BN, H), lambda i: (i, 0))
    one = pl.BlockSpec((1, H), lambda i: (0, 0))
    return pl.pallas_call(
        _nodeB_kernel,
        grid=(N // BN,),
        in_specs=[blk, one, one, blk, one, one],
        out_specs=blk,
        out_shape=jax.ShapeDtypeStruct((N, H), jnp.float32),
    )(out, cs, cq, x, bw.reshape(1, H), bb.reshape(1, H))


def _em_kernel(ga_ref, gb_ref, ea_ref, B3_ref, b1_ref, W2_ref, b2_ref, o_ref):
    hid = (ga_ref[...] + gb_ref[...]
           + _dot(ea_ref[...], B3_ref[...]) + b1_ref[...])
    em = _dot(jax.nn.relu(hid), W2_ref[...]) + b2_ref[...]
    o_ref[...] = ea_ref[...] + em * 0.5


def _em_edges(g, off_a, off_b, ea, B3, b1, W2, b2):
    KB = E // BE
    return pl.pallas_call(
        _em_kernel,
        grid=(KB,),
        in_specs=[
            pl.BlockSpec((BE, H), lambda i, o=off_a: (i + o * KB, 0)),
            pl.BlockSpec((BE, H), lambda i, o=off_b: (i + o * KB, 0)),
            pl.BlockSpec((BE, H), lambda i: (i, 0)),
            pl.BlockSpec((H, H), lambda i: (0, 0)),
            pl.BlockSpec((1, H), lambda i: (0, 0)),
            pl.BlockSpec((H, H), lambda i: (0, 0)),
            pl.BlockSpec((1, H), lambda i: (0, 0)),
        ],
        out_specs=pl.BlockSpec((BE, H), lambda i: (i, 0)),
        out_shape=jax.ShapeDtypeStruct((E, H), jnp.float32),
    )(g, g, ea, B3, b1.reshape(1, H), W2, b2.reshape(1, H))


def kernel(x, edge_index, edge_attr, node_W, node_b, edge_W, edge_b, ee_W,
           ee_b, pre_W, pre_b, post_W, post_b, lin_W, lin_b, em1_W, em1_b,
           em2_W, em2_b, bn_w, bn_b):
    src = edge_index[0]
    dst = edge_index[1]
    dst3 = dst.reshape(E // BS, 1, BS)

    W3p, bf = _prep(ee_W, ee_b, pre_W, pre_b)

    xc = _enc_nodes(x, node_W, node_b)
    ea = _enc_edges(edge_attr, edge_W, edge_b)

    W1 = [pre_W[i][:H] for i in range(2)]
    W2 = [pre_W[i][H:2 * H] for i in range(2)]
    B1 = [em1_W[i][:H] for i in range(2)]
    B2 = [em1_W[i][H:2 * H] for i in range(2)]
    B3 = [em1_W[i][2 * H:] for i in range(2)]
    PW = [post_W[i].reshape(13, H, H) for i in range(2)]

    idx_ds = jnp.concatenate([dst, src + N])
    idx_emn = jnp.concatenate([src, dst + N, dst + 2 * N, src + 3 * N])
    idx_em = jnp.concatenate([src, dst + N])

    cp = _scatter_add(jnp.ones((E, H), jnp.float32), dst)
    for i in range(2):
        if i == 0:
            T = _tables(xc, jnp.stack([W1[0], W2[0]]))
            g = _gather(T, idx_ds)
            m, m2 = _m_edges(g, ea, W3p[i], bf[i])
        sp = _scatter_add(m, dst)
        qp = _scatter_add(m2, dst)
        mm = _minmax(m, dst3)
        out, cs, cq = _nodeA(sp, qp, cp, mm, xc, PW[i], post_b[i],
                             lin_W[i], lin_b[i])
        xc = _nodeB(out, cs, cq, xc, bn_w[i], bn_b[i])
        if i == 0:
            T = _tables(xc, jnp.stack([B1[0], B2[0], W1[1], W2[1]]))
            g = _gather(T, idx_emn)
            ea = _em_edges(g, 0, 1, ea, B3[0], em1_b[0], em2_W[0], em2_b[0])
            m, m2 = _m_edges(g[2 * E:], ea, W3p[1], bf[1])
        else:
            T = _tables(xc, jnp.stack([B1[1], B2[1]]))
            g = _gather(T, idx_em)
            ea = _em_edges(g, 0, 1, ea, B3[1], em1_b[1], em2_W[1], em2_b[1])
    return xc, ea
